# baseline (speedup 1.0000x reference)
import jax
import jax.numpy as jnp
from jax.experimental import pallas as pl
from jax.experimental.pallas import tpu as pltpu

_FCHUNK = 512


def _mlp_kernel(x_ref, w1_ref, b1_ref, w2_ref, b2_ref, o_ref):
    xv = x_ref[0]
    F = w1_ref.shape[2]
    o = None
    for c in range(0, F, _FCHUNK):
        h = jnp.dot(xv, w1_ref[0, :, c:c + _FCHUNK],
                    preferred_element_type=jnp.float32)
        h = jnp.maximum(h + b1_ref[0, :, c:c + _FCHUNK], 0.0)
        part = jnp.dot(h, w2_ref[0, c:c + _FCHUNK, :],
                       preferred_element_type=jnp.float32)
        o = part if o is None else o + part
    o_ref[0] = o + b2_ref[0]


def kernel(x, W1, b1, W2, b2):
    B, S, D = x.shape
    E, _, F = W1.shape
    T = B * S
    per = T // E
    xr = x.reshape(E, per, D)
    out = pl.pallas_call(
        _mlp_kernel,
        grid=(E,),
        in_specs=[
            pl.BlockSpec((1, per, D), lambda e: (e, 0, 0)),
            pl.BlockSpec((1, D, F), lambda e: (e, 0, 0)),
            pl.BlockSpec((1, 1, F), lambda e: (e, 0, 0)),
            pl.BlockSpec((1, F, D), lambda e: (e, 0, 0)),
            pl.BlockSpec((1, 1, D), lambda e: (e, 0, 0)),
        ],
        out_specs=pl.BlockSpec((1, per, D), lambda e: (e, 0, 0)),
        out_shape=jax.ShapeDtypeStruct((E, per, D), x.dtype),
        compiler_params=pltpu.CompilerParams(
            dimension_semantics=("arbitrary",),
            vmem_limit_bytes=112 * 1024 * 1024,
        ),
    )(xr, W1, b1.reshape(E, 1, F), W2, b2.reshape(E, 1, D))
    return out.reshape(B, S, D)
